# baseline (speedup 1.0000x reference)
import functools

import jax
import jax.numpy as jnp
from jax import lax
from jax.experimental import pallas as pl
from jax.experimental.pallas import tpu as pltpu
from jax.experimental.pallas import tpu_sc as plsc

_NUM_CORES = 2
_NUM_SUBCORES = 16
_NUM_WORKERS = _NUM_CORES * _NUM_SUBCORES
_LANES = 16


def _sc_gather_sum(idx, table_t, n_valid, rpw, tail_start):
    emb, vocab = table_t.shape
    mesh = plsc.VectorSubcoreMesh(core_axis_name="c", subcore_axis_name="s")

    @functools.partial(
        pl.kernel,
        mesh=mesh,
        out_type=jax.ShapeDtypeStruct((_NUM_WORKERS, emb), jnp.float32),
        compiler_params=pltpu.CompilerParams(
            use_tc_tiling_on_sc=True, needs_layout_passes=False),
        scratch_types=[
            pltpu.VMEM((_LANES,), jnp.int32),
            pltpu.VMEM((rpw, emb, 128), jnp.float32),
            pltpu.VMEM((emb,), jnp.float32),
            pltpu.SemaphoreType.DMA,
        ],
    )
    def gather_sum(idx_hbm, table_hbm, out_hbm, idx_v, blocks_v, acc_v, sem):
        wid = lax.axis_index("s") * _NUM_CORES + lax.axis_index("c")
        base = wid * rpw
        nload = min(rpw, 8)
        pltpu.sync_copy(idx_hbm.at[pl.ds(base, nload)],
                        idx_v.at[pl.ds(0, nload)])
        v16 = jnp.clip(idx_v[...], 0, vocab - 1)
        lane = lax.iota(jnp.int32, _LANES)
        offs, copies, uses = [], [], []
        for j in range(rpw):
            idx_j = jnp.sum(jnp.where(lane == j, v16, 0))
            use_j = jnp.logical_and(base + j < n_valid, idx_j < tail_start)
            start_j = jnp.where(use_j, (idx_j >> 7) << 7, 0)
            offs.append(idx_j & 127)
            uses.append(use_j)
            cp = pltpu.make_async_copy(
                table_hbm.at[:, pl.ds(pl.multiple_of(start_j, 128), 128)],
                blocks_v.at[j], sem)
            copies.append(cp)

            @pl.when(use_j)
            def _(cp=cp):
                cp.start()

        for j in range(rpw):

            @pl.when(uses[j])
            def _(cp=copies[j]):
                cp.wait()

        accs = [jnp.zeros((_LANES,), jnp.float32) for _ in range(emb // _LANES)]
        for j in range(rpw):
            uv = jnp.broadcast_to(uses[j], (_LANES,))
            jv = jnp.full((_LANES,), j, jnp.int32)
            ov = jnp.broadcast_to(offs[j], (_LANES,))
            for c in range(emb // _LANES):
                dim = lane + c * _LANES
                v_main = plsc.load_gather(blocks_v, [jv, dim, ov])
                accs[c] = accs[c] + jnp.where(uv, v_main, 0.0)
        for c in range(emb // _LANES):
            acc_v[pl.ds(c * _LANES, _LANES)] = accs[c]
        pltpu.sync_copy(acc_v, out_hbm.at[wid])

    return gather_sum(idx, table_t)


def _tc_matvec(img_ref, wrow_ref, b_ref, out_ref):
    emb = wrow_ref.shape[1] - img_ref.shape[1]
    w_img = wrow_ref[:, emb:]
    scores = lax.dot_general(
        w_img, img_ref[...], (((1,), (1,)), ((), ())),
        preferred_element_type=jnp.float32)
    out_ref[...] = scores + b_ref[0, 0]


def _tc_combine(tail_start, scores_ref, part_ref, wrow_ref, slab_ref, idx_ref,
                out_ref):
    emb = part_ref.shape[1]
    w_emb = wrow_ref[:, :emb]
    s0 = jnp.sum(part_ref[...] * w_emb)
    ntail = slab_ref.shape[1]
    nidx = idx_ref.shape[1]
    idx_b = jnp.broadcast_to(idx_ref[...], (ntail, nidx))
    cols = lax.broadcasted_iota(jnp.int32, (ntail, nidx), 0) + tail_start
    hist = (idx_b == cols).astype(jnp.float32)
    t = lax.dot_general(w_emb, slab_ref[...], (((1,), (0,)), ((), ())),
                        preferred_element_type=jnp.float32)
    s_tail = jnp.sum(lax.dot_general(t, hist, (((1,), (0,)), ((), ())),
                                     preferred_element_type=jnp.float32))
    scores = scores_ref[...] + (s0 + s_tail)
    m = jnp.max(scores)
    e = jnp.exp(scores - m)
    out_ref[...] = e * (1.0 / jnp.sum(e))


def kernel(text_input, image_input, emb_table, W, b):
    out_dim, img_dim = image_input.shape
    vocab, emb = emb_table.shape

    idx = text_input.reshape(-1).astype(jnp.int32)
    n = idx.shape[0]
    rpw = -(-n // _NUM_WORKERS)
    rpw = -(-rpw // 8) * 8

    tail_start = (vocab // 128) * 128
    if tail_start == vocab:
        tail_start = vocab - 128

    table_t = emb_table.T
    partials = _sc_gather_sum(idx, table_t, n, rpw, tail_start)

    w_row = W.T
    b2 = b.reshape(1, 1)

    scores = pl.pallas_call(
        _tc_matvec,
        out_shape=jax.ShapeDtypeStruct((1, out_dim), jnp.float32),
    )(image_input, w_row, b2)

    slab_t = table_t[:, tail_start:]
    idx2 = idx.reshape(1, n)
    return pl.pallas_call(
        functools.partial(_tc_combine, tail_start),
        out_shape=jax.ShapeDtypeStruct((1, out_dim), jnp.float32),
    )(scores, partials, w_row, slab_t, idx2)
